# baseline (speedup 1.0000x reference)
import functools

import jax
import jax.numpy as jnp
from jax import lax
from jax.experimental import pallas as pl
from jax.experimental.pallas import tpu as pltpu
from jax.experimental.pallas import tpu_sc as plsc

_NC, _NS = 2, 16
_NW = _NC * _NS


def kernel(x, emb):
    b, s, d = x.shape
    n = s * d
    x2 = x.reshape(b, n)
    emb1 = emb[:s].reshape(n)
    rows_w = s // _NW
    ch_rows = 16
    ch = ch_rows * d
    n_ch = rows_w // ch_rows
    worker_elems = rows_w * d

    mesh = plsc.VectorSubcoreMesh(core_axis_name="c", subcore_axis_name="s")

    @functools.partial(
        pl.kernel,
        out_type=jax.ShapeDtypeStruct((b, n), jnp.float32),
        mesh=mesh,
        scratch_types=[
            pltpu.VMEM((ch,), jnp.float32),
            pltpu.VMEM((ch,), jnp.float32),
        ],
    )
    def sc_add(x_hbm, emb_hbm, out_hbm, emb_v, x_v):
        wid = lax.axis_index("s") * _NC + lax.axis_index("c")
        base = wid * worker_elems

        @pl.loop(0, n_ch)
        def _chunk(c):
            off = base + c * ch
            pltpu.sync_copy(emb_hbm.at[pl.ds(off, ch)], emb_v)
            for bb in range(b):
                pltpu.sync_copy(x_hbm.at[bb, pl.ds(off, ch)], x_v)

                @pl.loop(0, ch // 16, unroll=8)
                def _vec(i):
                    o = i * 16
                    x_v[pl.ds(o, 16)] = x_v[pl.ds(o, 16)] + emb_v[pl.ds(o, 16)]

                pltpu.sync_copy(x_v, out_hbm.at[bb, pl.ds(off, ch)])

    out = sc_add(x2, emb1)
    return out.reshape(b, s, d)


# speedup vs baseline: 1.1980x; 1.1980x over previous
import functools

import jax
import jax.numpy as jnp
from jax import lax
from jax.experimental import pallas as pl
from jax.experimental.pallas import tpu as pltpu
from jax.experimental.pallas import tpu_sc as plsc

_NC, _NS = 2, 16
_NW = _NC * _NS


def kernel(x, emb):
    b, s, d = x.shape
    n = s * d
    x2 = x.reshape(b, n)
    emb1 = emb[:s].reshape(n)
    rows_w = s // _NW
    ch_rows = 8
    ch = ch_rows * d
    n_ch = rows_w // ch_rows
    worker_elems = rows_w * d

    mesh = plsc.VectorSubcoreMesh(core_axis_name="c", subcore_axis_name="s")

    scratch = (
        [pltpu.VMEM((ch,), jnp.float32) for _ in range(2)]
        + [pltpu.VMEM((ch,), jnp.float32) for _ in range(b)]
        + [pltpu.VMEM((ch,), jnp.float32) for _ in range(b)]
        + [pltpu.SemaphoreType.DMA for _ in range(2 + 2 * b)]
    )

    @functools.partial(
        pl.kernel,
        out_type=jax.ShapeDtypeStruct((b, n), jnp.float32),
        mesh=mesh,
        scratch_types=scratch,
    )
    def sc_add(x_hbm, emb_hbm, out_hbm, *bufs):
        ev = bufs[0:2]
        xv = bufs[2:2 + b]
        ov = bufs[2 + b:2 + 2 * b]
        esem = bufs[2 + 2 * b:4 + 2 * b]
        xsem = bufs[4 + 2 * b:4 + 3 * b]
        osem = bufs[4 + 3 * b:4 + 4 * b]

        wid = lax.axis_index("s") * _NC + lax.axis_index("c")
        base = wid * worker_elems

        def off(c):
            return base + c * ch

        pltpu.async_copy(emb_hbm.at[pl.ds(off(0), ch)], ev[0], esem[0])
        pltpu.async_copy(emb_hbm.at[pl.ds(off(1), ch)], ev[1], esem[1])
        for j in range(b):
            pltpu.async_copy(x_hbm.at[j, pl.ds(off(0), ch)], xv[j], xsem[j])

        @pl.loop(0, n_ch, step=2)
        def _chunks(c0):
            for cc in range(2):
                c = c0 + cc
                for j in range(b):
                    pltpu.make_async_copy(
                        x_hbm.at[j, pl.ds(off(c), ch)], xv[j], xsem[j]
                    ).wait()
                    if j == 0:
                        pltpu.make_async_copy(
                            emb_hbm.at[pl.ds(off(c), ch)], ev[cc], esem[cc]
                        ).wait()
                    @pl.when(c > 0)
                    def _():
                        pltpu.make_async_copy(
                            ov[j], out_hbm.at[j, pl.ds(off(c), ch)], osem[j]
                        ).wait()

                    @pl.loop(0, ch // 16, unroll=8)
                    def _vec(i):
                        o = i * 16
                        ov[j][pl.ds(o, 16)] = (
                            xv[j][pl.ds(o, 16)] + ev[cc][pl.ds(o, 16)]
                        )

                    @pl.when(c + 1 < n_ch)
                    def _():
                        pltpu.async_copy(
                            x_hbm.at[j, pl.ds(off(c + 1), ch)], xv[j], xsem[j]
                        )

                    pltpu.async_copy(
                        ov[j], out_hbm.at[j, pl.ds(off(c), ch)], osem[j]
                    )
                @pl.when(c + 2 < n_ch)
                def _():
                    pltpu.async_copy(
                        emb_hbm.at[pl.ds(off(c + 2), ch)], ev[cc], esem[cc]
                    )

        for j in range(b):
            pltpu.make_async_copy(
                ov[j], out_hbm.at[j, pl.ds(off(n_ch - 1), ch)], osem[j]
            ).wait()

    out = sc_add(x2, emb1)
    return out.reshape(b, s, d)


# speedup vs baseline: 2.0493x; 1.7105x over previous
import functools

import jax
import jax.numpy as jnp
from jax import lax
from jax.experimental import pallas as pl
from jax.experimental.pallas import tpu as pltpu
from jax.experimental.pallas import tpu_sc as plsc

_NC, _NS = 2, 16
_NW = _NC * _NS


def kernel(x, emb):
    b, s, d = x.shape
    n = s * d
    x2 = x.reshape(b, n)
    emb1 = emb[:s].reshape(n)
    rows_w = s // _NW
    ch_rows = 8
    ch = ch_rows * d
    n_ch = rows_w // ch_rows
    worker_elems = rows_w * d

    mesh = plsc.VectorSubcoreMesh(core_axis_name="c", subcore_axis_name="s")

    scratch = (
        [pltpu.VMEM((ch,), jnp.float32) for _ in range(2)]
        + [pltpu.VMEM((ch,), jnp.float32) for _ in range(b)]
        + [pltpu.VMEM((ch,), jnp.float32) for _ in range(b)]
        + [pltpu.SemaphoreType.DMA for _ in range(2 + 2 * b)]
    )

    @functools.partial(
        pl.kernel,
        out_type=jax.ShapeDtypeStruct((b, n), jnp.float32),
        mesh=mesh,
        scratch_types=scratch,
    )
    def sc_add(x_hbm, emb_hbm, out_hbm, *bufs):
        ev = bufs[0:2]
        xv = bufs[2:2 + b]
        ov = bufs[2 + b:2 + 2 * b]
        esem = bufs[2 + 2 * b:4 + 2 * b]
        xsem = bufs[4 + 2 * b:4 + 3 * b]
        osem = bufs[4 + 3 * b:4 + 4 * b]

        wid = lax.axis_index("s") * _NC + lax.axis_index("c")
        base = wid * worker_elems

        def off(c):
            return base + c * ch

        pltpu.async_copy(emb_hbm.at[pl.ds(off(0), ch)], ev[0], esem[0])
        pltpu.async_copy(emb_hbm.at[pl.ds(off(1), ch)], ev[1], esem[1])
        for j in range(b):
            pltpu.async_copy(x_hbm.at[j, pl.ds(off(0), ch)], xv[j], xsem[j])

        @pl.loop(0, n_ch, step=2)
        def _chunks(c0):
            for cc in range(2):
                c = c0 + cc
                for j in range(b):
                    pltpu.make_async_copy(
                        x_hbm.at[j, pl.ds(off(c), ch)], xv[j], xsem[j]
                    ).wait()
                    if j == 0:
                        pltpu.make_async_copy(
                            emb_hbm.at[pl.ds(off(c), ch)], ev[cc], esem[cc]
                        ).wait()
                    @pl.when(c > 0)
                    def _():
                        pltpu.make_async_copy(
                            ov[j], out_hbm.at[j, pl.ds(off(c), ch)], osem[j]
                        ).wait()

                    @plsc.parallel_loop(0, ch, step=16, unroll=8)
                    def _vec(o):
                        ov[j][pl.ds(o, 16)] = (
                            xv[j][pl.ds(o, 16)] + ev[cc][pl.ds(o, 16)]
                        )

                    @pl.when(c + 1 < n_ch)
                    def _():
                        pltpu.async_copy(
                            x_hbm.at[j, pl.ds(off(c + 1), ch)], xv[j], xsem[j]
                        )

                    pltpu.async_copy(
                        ov[j], out_hbm.at[j, pl.ds(off(c), ch)], osem[j]
                    )
                @pl.when(c + 2 < n_ch)
                def _():
                    pltpu.async_copy(
                        emb_hbm.at[pl.ds(off(c + 2), ch)], ev[cc], esem[cc]
                    )

        for j in range(b):
            pltpu.make_async_copy(
                ov[j], out_hbm.at[j, pl.ds(off(n_ch - 1), ch)], osem[j]
            ).wait()

    out = sc_add(x2, emb1)
    return out.reshape(b, s, d)


# speedup vs baseline: 5.2713x; 2.5723x over previous
import functools

import jax
import jax.numpy as jnp
from jax import lax
from jax.experimental import pallas as pl
from jax.experimental.pallas import tpu as pltpu
from jax.experimental.pallas import tpu_sc as plsc

_NC, _NS = 2, 16
_NW = _NC * _NS


def kernel(x, emb):
    b, s, d = x.shape
    pe = emb[:s]
    rows_w = s // _NW
    ch_rows = 8
    n_ch = rows_w // ch_rows

    mesh = plsc.VectorSubcoreMesh(core_axis_name="c", subcore_axis_name="s")

    scratch = (
        [pltpu.VMEM((ch_rows, d), jnp.float32) for _ in range(2)]
        + [pltpu.VMEM((ch_rows, d), jnp.float32) for _ in range(b)]
        + [pltpu.VMEM((ch_rows, d), jnp.float32) for _ in range(b)]
        + [pltpu.SemaphoreType.DMA for _ in range(2 + 2 * b)]
    )

    @functools.partial(
        pl.kernel,
        out_type=jax.ShapeDtypeStruct((b, s, d), jnp.float32),
        mesh=mesh,
        scratch_types=scratch,
    )
    def sc_add(x_hbm, emb_hbm, out_hbm, *bufs):
        ev = bufs[0:2]
        xv = bufs[2:2 + b]
        ov = bufs[2 + b:2 + 2 * b]
        esem = bufs[2 + 2 * b:4 + 2 * b]
        xsem = bufs[4 + 2 * b:4 + 3 * b]
        osem = bufs[4 + 3 * b:4 + 4 * b]

        wid = lax.axis_index("s") * _NC + lax.axis_index("c")
        base = wid * rows_w

        def row(c):
            return base + c * ch_rows

        pltpu.async_copy(emb_hbm.at[pl.ds(row(0), ch_rows)], ev[0], esem[0])
        pltpu.async_copy(emb_hbm.at[pl.ds(row(1), ch_rows)], ev[1], esem[1])
        for j in range(b):
            pltpu.async_copy(x_hbm.at[j, pl.ds(row(0), ch_rows)], xv[j], xsem[j])

        @pl.loop(0, n_ch, step=2)
        def _chunks(c0):
            for cc in range(2):
                c = c0 + cc
                for j in range(b):
                    pltpu.make_async_copy(
                        x_hbm.at[j, pl.ds(row(c), ch_rows)], xv[j], xsem[j]
                    ).wait()
                    if j == 0:
                        pltpu.make_async_copy(
                            emb_hbm.at[pl.ds(row(c), ch_rows)], ev[cc], esem[cc]
                        ).wait()
                    @pl.when(c > 0)
                    def _():
                        pltpu.make_async_copy(
                            ov[j], out_hbm.at[j, pl.ds(row(c), ch_rows)], osem[j]
                        ).wait()

                    for r in range(ch_rows):
                        @plsc.parallel_loop(0, d, step=16, unroll=8)
                        def _vec(o):
                            ov[j][r, pl.ds(o, 16)] = (
                                xv[j][r, pl.ds(o, 16)] + ev[cc][r, pl.ds(o, 16)]
                            )

                    @pl.when(c + 1 < n_ch)
                    def _():
                        pltpu.async_copy(
                            x_hbm.at[j, pl.ds(row(c + 1), ch_rows)], xv[j], xsem[j]
                        )

                    pltpu.async_copy(
                        ov[j], out_hbm.at[j, pl.ds(row(c), ch_rows)], osem[j]
                    )
                @pl.when(c + 2 < n_ch)
                def _():
                    pltpu.async_copy(
                        emb_hbm.at[pl.ds(row(c + 2), ch_rows)], ev[cc], esem[cc]
                    )

        for j in range(b):
            pltpu.make_async_copy(
                ov[j], out_hbm.at[j, pl.ds(row(n_ch - 1), ch_rows)], osem[j]
            ).wait()

    return sc_add(x, pe)
